# baseline (speedup 1.0000x reference)
import jax
import jax.numpy as jnp
from jax.experimental import pallas as pl
from jax.experimental.pallas import tpu as pltpu

_NUM_EXPERTS = 8
_TOP_K = 2
_CHUNK_ROWS = 256
_DEPTH = 8
_LAG = 3


def _router_chunk(x, w):
    logits = jnp.dot(x, w, preferred_element_type=jnp.float32)
    m = jnp.max(logits, axis=-1, keepdims=True)
    e = jnp.exp(logits - m)
    probs = e / jnp.sum(e, axis=-1, keepdims=True)
    iota = jax.lax.broadcasted_iota(jnp.int32, probs.shape, 1)
    g1 = jnp.max(probs, axis=-1, keepdims=True)
    i1 = jnp.min(jnp.where(probs == g1, iota, _NUM_EXPERTS), axis=-1,
                 keepdims=True)
    rest = jnp.where(iota == i1, -jnp.inf, probs)
    g2 = jnp.max(rest, axis=-1, keepdims=True)
    i2 = jnp.min(jnp.where(rest == g2, iota, _NUM_EXPERTS), axis=-1,
                 keepdims=True)
    return jnp.concatenate(
        [g1, g2, i1.astype(jnp.float32), i2.astype(jnp.float32)], axis=-1)


def _moe_pipeline(x_hbm, w_ref, out_hbm, route_ref, buf, in_sem, out_sem):
    n = x_hbm.shape[0] // _CHUNK_ROWS
    w = w_ref[...]

    def in_copy(i, slot):
        return pltpu.make_async_copy(
            x_hbm.at[pl.ds(i * _CHUNK_ROWS, _CHUNK_ROWS), :],
            buf.at[slot], in_sem)

    def out_copy(i, slot):
        return pltpu.make_async_copy(
            buf.at[slot],
            out_hbm.at[pl.ds(i * _CHUNK_ROWS, _CHUNK_ROWS), :], out_sem)

    for i in range(n + _LAG):
        if i < n:
            slot = i % _DEPTH
            if i >= _DEPTH:
                out_copy(i - _DEPTH, slot).wait()
            in_copy(i, slot).start()
        j = i - _LAG
        if 0 <= j < n:
            slot = j % _DEPTH
            in_copy(j, slot).wait()
            out_copy(j, slot).start()
            route_ref[pl.ds(j * _CHUNK_ROWS, _CHUNK_ROWS), :] = _router_chunk(
                buf[slot], w)
    for j in range(max(0, n - _DEPTH), n):
        out_copy(j, j % _DEPTH).wait()


def kernel(inputs, W):
    b, s, d = inputs.shape
    n_tokens = b * s
    x = inputs.reshape(n_tokens, d)
    out, _ = pl.pallas_call(
        _moe_pipeline,
        in_specs=[
            pl.BlockSpec(memory_space=pl.ANY),
            pl.BlockSpec((d, _NUM_EXPERTS), lambda: (0, 0)),
        ],
        out_specs=[
            pl.BlockSpec(memory_space=pl.ANY),
            pl.BlockSpec((n_tokens, 2 * _TOP_K), lambda: (0, 0)),
        ],
        out_shape=[
            jax.ShapeDtypeStruct((n_tokens, d), jnp.float32),
            jax.ShapeDtypeStruct((n_tokens, 2 * _TOP_K), jnp.float32),
        ],
        scratch_shapes=[
            pltpu.VMEM((_DEPTH, _CHUNK_ROWS, d), jnp.float32),
            pltpu.SemaphoreType.DMA,
            pltpu.SemaphoreType.DMA,
        ],
    )(x, W)
    return out.reshape(inputs.shape)


# speedup vs baseline: 1.0104x; 1.0104x over previous
import jax
import jax.numpy as jnp
from jax.experimental import pallas as pl
from jax.experimental.pallas import tpu as pltpu

_NUM_EXPERTS = 8
_TOP_K = 2
_CHUNK_ROWS = 512
_DEPTH = 4
_LAG = 1


def _router_chunk(x, w):
    logits = jnp.dot(x, w, preferred_element_type=jnp.float32)
    m = jnp.max(logits, axis=-1, keepdims=True)
    e = jnp.exp(logits - m)
    probs = e / jnp.sum(e, axis=-1, keepdims=True)
    iota = jax.lax.broadcasted_iota(jnp.int32, probs.shape, 1)
    g1 = jnp.max(probs, axis=-1, keepdims=True)
    i1 = jnp.min(jnp.where(probs == g1, iota, _NUM_EXPERTS), axis=-1,
                 keepdims=True)
    rest = jnp.where(iota == i1, -jnp.inf, probs)
    g2 = jnp.max(rest, axis=-1, keepdims=True)
    i2 = jnp.min(jnp.where(rest == g2, iota, _NUM_EXPERTS), axis=-1,
                 keepdims=True)
    return jnp.concatenate(
        [g1, g2, i1.astype(jnp.float32), i2.astype(jnp.float32)], axis=-1)


def _moe_pipeline(x_hbm, w_ref, out_hbm, route_ref, buf, in_sem, out_sem):
    n = x_hbm.shape[0] // _CHUNK_ROWS
    w = w_ref[...]

    def in_copy(i, slot):
        return pltpu.make_async_copy(
            x_hbm.at[pl.ds(i * _CHUNK_ROWS, _CHUNK_ROWS), :],
            buf.at[slot], in_sem)

    def out_copy(i, slot):
        return pltpu.make_async_copy(
            buf.at[slot],
            out_hbm.at[pl.ds(i * _CHUNK_ROWS, _CHUNK_ROWS), :], out_sem)

    for i in range(n + _LAG):
        if i < n:
            slot = i % _DEPTH
            if i >= _DEPTH:
                out_copy(i - _DEPTH, slot).wait()
            in_copy(i, slot).start()
        j = i - _LAG
        if 0 <= j < n:
            slot = j % _DEPTH
            in_copy(j, slot).wait()
            out_copy(j, slot).start()
            route_ref[pl.ds(j * _CHUNK_ROWS, _CHUNK_ROWS), :] = _router_chunk(
                buf[slot], w)
    for j in range(max(0, n - _DEPTH), n):
        out_copy(j, j % _DEPTH).wait()


def kernel(inputs, W):
    b, s, d = inputs.shape
    n_tokens = b * s
    x = inputs.reshape(n_tokens, d)
    out, _ = pl.pallas_call(
        _moe_pipeline,
        in_specs=[
            pl.BlockSpec(memory_space=pl.ANY),
            pl.BlockSpec((d, _NUM_EXPERTS), lambda: (0, 0)),
        ],
        out_specs=[
            pl.BlockSpec(memory_space=pl.ANY),
            pl.BlockSpec((n_tokens, 2 * _TOP_K), lambda: (0, 0)),
        ],
        out_shape=[
            jax.ShapeDtypeStruct((n_tokens, d), jnp.float32),
            jax.ShapeDtypeStruct((n_tokens, 2 * _TOP_K), jnp.float32),
        ],
        scratch_shapes=[
            pltpu.VMEM((_DEPTH, _CHUNK_ROWS, d), jnp.float32),
            pltpu.SemaphoreType.DMA,
            pltpu.SemaphoreType.DMA,
        ],
    )(x, W)
    return out.reshape(inputs.shape)


# speedup vs baseline: 1.0215x; 1.0111x over previous
import jax
import jax.numpy as jnp
from jax.experimental import pallas as pl
from jax.experimental.pallas import tpu as pltpu

_NUM_EXPERTS = 8
_TOP_K = 2
_BLOCK_ROWS = 512


def _router_chunk(x, w):
    logits = jnp.dot(x, w, preferred_element_type=jnp.float32)
    m = jnp.max(logits, axis=-1, keepdims=True)
    e = jnp.exp(logits - m)
    probs = e / jnp.sum(e, axis=-1, keepdims=True)
    iota = jax.lax.broadcasted_iota(jnp.int32, probs.shape, 1)
    g1 = jnp.max(probs, axis=-1, keepdims=True)
    i1 = jnp.min(jnp.where(probs == g1, iota, _NUM_EXPERTS), axis=-1,
                 keepdims=True)
    rest = jnp.where(iota == i1, -jnp.inf, probs)
    g2 = jnp.max(rest, axis=-1, keepdims=True)
    i2 = jnp.min(jnp.where(rest == g2, iota, _NUM_EXPERTS), axis=-1,
                 keepdims=True)
    return jnp.concatenate(
        [g1, g2, i1.astype(jnp.float32), i2.astype(jnp.float32)], axis=-1)


def _moe_block(x_ref, w_ref, out_ref, route_ref):
    x = x_ref[...]
    out_ref[...] = x
    route_ref[...] = _router_chunk(x, w_ref[...])


def kernel(inputs, W):
    b, s, d = inputs.shape
    n_tokens = b * s
    x = inputs.reshape(n_tokens, d)
    grid = (n_tokens // _BLOCK_ROWS,)
    out, _ = pl.pallas_call(
        _moe_block,
        grid=grid,
        in_specs=[
            pl.BlockSpec((_BLOCK_ROWS, d), lambda i: (i, 0)),
            pl.BlockSpec((d, _NUM_EXPERTS), lambda i: (0, 0)),
        ],
        out_specs=[
            pl.BlockSpec((_BLOCK_ROWS, d), lambda i: (i, 0)),
            pl.BlockSpec((_BLOCK_ROWS, 2 * _TOP_K), lambda i: (i, 0)),
        ],
        out_shape=[
            jax.ShapeDtypeStruct((n_tokens, d), jnp.float32),
            jax.ShapeDtypeStruct((n_tokens, 2 * _TOP_K), jnp.float32),
        ],
        compiler_params=pltpu.CompilerParams(
            dimension_semantics=("parallel",)),
    )(x, W)
    return out.reshape(inputs.shape)
